# baseline (speedup 1.0000x reference)
import functools

import jax
import jax.numpy as jnp
from jax import lax
from jax.experimental import pallas as pl
from jax.experimental.pallas import tpu as pltpu
from jax.experimental.pallas import tpu_sc as plsc

NUM_USERS = 1000000
NUM_ITEMS = 1000000
EMB = 16
B = 16384

_info = plsc.get_sparse_core_info()
NC = _info.num_cores
NS = _info.num_subcores
L = _info.num_lanes
NW = NC * NS
BPW = B // NW

_mesh = plsc.VectorSubcoreMesh(core_axis_name="c", subcore_axis_name="s")


@functools.partial(
    pl.kernel,
    mesh=_mesh,
    out_type=jax.ShapeDtypeStruct((B,), jnp.float32),
    scratch_types=[
        pltpu.VMEM((BPW,), jnp.int32),
        pltpu.VMEM((BPW,), jnp.int32),
        pltpu.VMEM((BPW, EMB), jnp.float32),
        pltpu.VMEM((BPW, EMB), jnp.float32),
        pltpu.VMEM((BPW * EMB,), jnp.float32),
        pltpu.VMEM((BPW,), jnp.float32),
        pltpu.SemaphoreType.DMA,
        pltpu.SemaphoreType.DMA,
    ],
    compiler_params=pltpu.CompilerParams(
        needs_layout_passes=False, use_tc_tiling_on_sc=False),
)
def _sc_predict(uid_hbm, iid_hbm, ue_hbm, ie_hbm, out_hbm,
                uid_v, iid_v, u_v, v_v, prod_v, p_v, sem_u, sem_i):
    wid = lax.axis_index("s") * NC + lax.axis_index("c")
    base = wid * BPW
    pltpu.sync_copy(uid_hbm.at[pl.ds(base, BPW)], uid_v)
    pltpu.sync_copy(iid_hbm.at[pl.ds(base, BPW)], iid_v)
    cp_u = pltpu.async_copy(ue_hbm.at[uid_v], u_v, sem_u)
    cp_i = pltpu.async_copy(ie_hbm.at[iid_v], v_v, sem_i)
    cp_u.wait()
    cp_i.wait()

    lane = lax.iota(jnp.int32, L)

    def body(i, carry):
        for j in range(L):
            r = i * L + j
            prod_v[pl.ds(r * EMB, EMB)] = u_v[r, :] * v_v[r, :]
        acc = jnp.zeros((L,), jnp.float32)
        base_idx = i * (L * EMB) + lane * EMB
        for k in range(EMB):
            acc = acc + plsc.load_gather(prod_v, [base_idx + k])
        p_v[pl.ds(i * L, L)] = acc
        return carry

    lax.fori_loop(0, BPW // L, body, 0)
    pltpu.sync_copy(p_v, out_hbm.at[pl.ds(base, BPW)])


def _tc_loss_body(pred_ref, y_ref, out_ref):
    d = pred_ref[...] - y_ref[...]
    out_ref[0, 0] = jnp.sum(d * d) * (1.0 / B)


_tc_loss = pl.pallas_call(
    _tc_loss_body,
    out_shape=jax.ShapeDtypeStruct((1, 1), jnp.float32),
    out_specs=pl.BlockSpec(memory_space=pltpu.SMEM),
)


def kernel(uid, iid, y, user_emb, item_emb):
    uid = uid.astype(jnp.int32)
    iid = iid.astype(jnp.int32)
    pred = _sc_predict(uid, iid, user_emb, item_emb)
    loss = _tc_loss(pred.reshape(128, 128), y.reshape(128, 128))[0, 0]
    return pred, loss


# speedup vs baseline: 5.9759x; 5.9759x over previous
import functools

import jax
import jax.numpy as jnp
from jax import lax
from jax.experimental import pallas as pl
from jax.experimental.pallas import tpu as pltpu
from jax.experimental.pallas import tpu_sc as plsc

NUM_USERS = 1000000
NUM_ITEMS = 1000000
EMB = 16
B = 16384

_info = plsc.get_sparse_core_info()
NC = _info.num_cores
NS = _info.num_subcores
L = _info.num_lanes
NW = NC * NS
BPW = B // NW
G = 16

_mesh = plsc.VectorSubcoreMesh(core_axis_name="c", subcore_axis_name="s")


@functools.partial(
    pl.kernel,
    mesh=_mesh,
    out_type=jax.ShapeDtypeStruct((B,), jnp.float32),
    scratch_types=[
        pltpu.VMEM((BPW,), jnp.int32),
        pltpu.VMEM((BPW,), jnp.int32),
        pltpu.VMEM((G, 2, 8, 128), jnp.float32),
        pltpu.VMEM((G, 2, 8, 128), jnp.float32),
        pltpu.VMEM((BPW,), jnp.float32),
        pltpu.SemaphoreType.DMA,
    ],
    compiler_params=pltpu.CompilerParams(
        needs_layout_passes=False, use_tc_tiling_on_sc=True),
)
def _sc_predict(uid_hbm, iid_hbm, ue_hbm, ie_hbm, out_hbm,
                uid_v, iid_v, ub_v, vb_v, p_v, sem):
    wid = lax.axis_index("s") * NC + lax.axis_index("c")
    base = wid * BPW
    pltpu.sync_copy(uid_hbm.at[pl.ds(base, BPW)], uid_v)
    pltpu.sync_copy(iid_hbm.at[pl.ds(base, BPW)], iid_v)

    lane = lax.iota(jnp.int32, L)

    def group(gi, carry):
        ids_u = uid_v[pl.ds(gi * G, G)]
        ids_i = iid_v[pl.ds(gi * G, G)]
        wus = (ids_u // 128) * 128
        wis = (ids_i // 128) * 128
        for g in range(G):
            wu = pl.multiple_of(wus[g], 128)
            wi = pl.multiple_of(wis[g], 128)
            pltpu.async_copy(
                ue_hbm.at[:, :, pl.ds(wu, 128)], ub_v.at[g], sem)
            pltpu.async_copy(
                ie_hbm.at[:, :, pl.ds(wi, 128)], vb_v.at[g], sem)
        for g in range(2 * G):
            pltpu.make_async_copy(
                ue_hbm.at[:, :, pl.ds(0, 128)], ub_v.at[0], sem).wait()
        cu = lax.rem(ids_u, 128)
        ci = lax.rem(ids_i, 128)
        acc = jnp.zeros((L,), jnp.float32)
        for j in range(EMB):
            jhi = jnp.full((L,), j // 8, jnp.int32)
            jlo = jnp.full((L,), j % 8, jnp.int32)
            uu = plsc.load_gather(ub_v, [lane, jhi, jlo, cu])
            vv = plsc.load_gather(vb_v, [lane, jhi, jlo, ci])
            acc = acc + uu * vv
        p_v[pl.ds(gi * G, G)] = acc
        return carry

    lax.fori_loop(0, BPW // G, group, 0)
    pltpu.sync_copy(p_v, out_hbm.at[pl.ds(base, BPW)])


def _tc_loss_body(pred_ref, y_ref, out_ref):
    d = pred_ref[...] - y_ref[...]
    out_ref[0, 0] = jnp.sum(d * d) * (1.0 / B)


_tc_loss = pl.pallas_call(
    _tc_loss_body,
    out_shape=jax.ShapeDtypeStruct((1, 1), jnp.float32),
    out_specs=pl.BlockSpec(memory_space=pltpu.SMEM),
)


def kernel(uid, iid, y, user_emb, item_emb):
    uid = uid.astype(jnp.int32)
    iid = iid.astype(jnp.int32)
    ue3 = user_emb.T.reshape(2, 8, NUM_USERS)
    ie3 = item_emb.T.reshape(2, 8, NUM_ITEMS)
    pred = _sc_predict(uid, iid, ue3, ie3)
    loss = _tc_loss(pred.reshape(128, 128), y.reshape(128, 128))[0, 0]
    return pred, loss


# speedup vs baseline: 11.4574x; 1.9173x over previous
import functools

import jax
import jax.numpy as jnp
from jax import lax
from jax.experimental import pallas as pl
from jax.experimental.pallas import tpu as pltpu
from jax.experimental.pallas import tpu_sc as plsc

NUM_USERS = 1000000
NUM_ITEMS = 1000000
EMB = 16
B = 16384

_info = plsc.get_sparse_core_info()
NC = _info.num_cores
NS = _info.num_subcores
L = _info.num_lanes
NW = NC * NS
BPW = B // NW
G = 16

_mesh = plsc.VectorSubcoreMesh(core_axis_name="c", subcore_axis_name="s")


@functools.partial(
    pl.kernel,
    mesh=_mesh,
    out_type=jax.ShapeDtypeStruct((B,), jnp.float32),
    scratch_types=[
        pltpu.VMEM((BPW,), jnp.int32),
        pltpu.VMEM((BPW,), jnp.int32),
        pltpu.VMEM((G, 2, 8, 128), jnp.float32),
        pltpu.VMEM((G, 2, 8, 128), jnp.float32),
        pltpu.VMEM((BPW,), jnp.float32),
        pltpu.SemaphoreType.DMA,
    ],
    compiler_params=pltpu.CompilerParams(
        needs_layout_passes=False, use_tc_tiling_on_sc=True),
)
def _sc_predict(uid_hbm, iid_hbm, ue_hbm, ie_hbm, out_hbm,
                uid_v, iid_v, ub_v, vb_v, p_v, sem):
    wid = lax.axis_index("s") * NC + lax.axis_index("c")
    base = wid * BPW
    pltpu.sync_copy(uid_hbm.at[pl.ds(base, BPW)], uid_v)
    pltpu.sync_copy(iid_hbm.at[pl.ds(base, BPW)], iid_v)

    lane = lax.iota(jnp.int32, L)

    def group(gi, carry):
        ids_u = uid_v[pl.ds(gi * G, G)]
        ids_i = iid_v[pl.ds(gi * G, G)]
        wus = (ids_u // G) * G
        wis = (ids_i // G) * G
        for g in range(G):
            wu = pl.multiple_of(wus[g], 16)
            wi = pl.multiple_of(wis[g], 16)
            pltpu.async_copy(
                ue_hbm.at[:, :, pl.ds(wu, G)],
                ub_v.at[g, :, :, pl.ds(0, G)], sem)
            pltpu.async_copy(
                ie_hbm.at[:, :, pl.ds(wi, G)],
                vb_v.at[g, :, :, pl.ds(0, G)], sem)
        for g in range(2 * G):
            pltpu.make_async_copy(
                ue_hbm.at[:, :, pl.ds(0, G)],
                ub_v.at[0, :, :, pl.ds(0, G)], sem).wait()
        cu = lax.rem(ids_u, G)
        ci = lax.rem(ids_i, G)
        acc = jnp.zeros((L,), jnp.float32)
        for j in range(EMB):
            jhi = jnp.full((L,), j // 8, jnp.int32)
            jlo = jnp.full((L,), j % 8, jnp.int32)
            uu = plsc.load_gather(ub_v, [lane, jhi, jlo, cu])
            vv = plsc.load_gather(vb_v, [lane, jhi, jlo, ci])
            acc = acc + uu * vv
        p_v[pl.ds(gi * G, G)] = acc
        return carry

    lax.fori_loop(0, BPW // G, group, 0)
    pltpu.sync_copy(p_v, out_hbm.at[pl.ds(base, BPW)])


def _tc_loss_body(pred_ref, y_ref, out_ref):
    d = pred_ref[...] - y_ref[...]
    out_ref[0, 0] = jnp.sum(d * d) * (1.0 / B)


_tc_loss = pl.pallas_call(
    _tc_loss_body,
    out_shape=jax.ShapeDtypeStruct((1, 1), jnp.float32),
    out_specs=pl.BlockSpec(memory_space=pltpu.SMEM),
)


def kernel(uid, iid, y, user_emb, item_emb):
    uid = uid.astype(jnp.int32)
    iid = iid.astype(jnp.int32)
    ue3 = user_emb.T.reshape(2, 8, NUM_USERS)
    ie3 = item_emb.T.reshape(2, 8, NUM_ITEMS)
    pred = _sc_predict(uid, iid, ue3, ie3)
    loss = _tc_loss(pred.reshape(128, 128), y.reshape(128, 128))[0, 0]
    return pred, loss
